# baseline (speedup 1.0000x reference)
import functools
import math

import jax
import jax.numpy as jnp
from jax.experimental import pallas as pl
from jax.experimental.pallas import tpu as pltpu

_P = 24
_D = 384
_H = 8
_HD = 24
_HC = _H * _HD
_M = 16
_N = _P * _P
_HID = _D * 4
_HM = _H * _M


def _ln(x, g, b):
    rvec = jnp.full((_D, 8), 1.0 / _D, jnp.float32)
    mu = jax.lax.dot_general(x, rvec, (((1,), (0,)), ((), ())),
                             preferred_element_type=jnp.float32)[:, :1]
    ms = jax.lax.dot_general(x * x, rvec, (((1,), (0,)), ((), ())),
                             preferred_element_type=jnp.float32)[:, :1]
    var = jnp.maximum(ms - mu * mu, 0.0)
    return (x - mu) * jax.lax.rsqrt(var + 1e-5) * g + b


def _dot_t(a, b):
    return jax.lax.dot_general(a, b, (((1,), (1,)), ((), ())),
                               preferred_element_type=jnp.float32)


def _dot(a, b):
    return jax.lax.dot_general(a, b, (((1,), (0,)), ((), ())),
                               preferred_element_type=jnp.float32)


def _fused(x_ref, g1_ref, be1_ref, g2_ref, be2_ref, Wf_ref, bf_ref,
           Wv_ref, bv_ref, sa_ref, sb_ref, Wp_ref, bp_ref,
           W1_ref, b1_ref, W2_ref, b2_ref, o_ref, *, blk):
    f32 = jnp.float32
    ncol = jax.lax.broadcasted_iota(jnp.int32, (_M, _N), 1)
    mrow = jax.lax.broadcasted_iota(jnp.int32, (_M, _N), 0)
    p1 = ncol // _P
    p2 = ncol - p1 * _P
    mid = (p1 // 6) * 4 + (p2 // 6)
    Amat = jnp.where(mid == mrow, f32(1.0 / 36.0), f32(0.0))
    r0 = jax.lax.broadcasted_iota(jnp.int32, (_HC, _HC), 0)
    c0 = jax.lax.broadcasted_iota(jnp.int32, (_HC, _HC), 1)
    BD = jnp.where(r0 // _HD == c0 // _HD, f32(1.0), f32(0.0))
    rowh = jax.lax.broadcasted_iota(jnp.int32, (_HM, _HC), 0) // _M
    colh = jax.lax.broadcasted_iota(jnp.int32, (_HM, _HC), 1) // _HD
    bmask = jnp.where(rowh == colh, f32(1.0), f32(0.0))
    miota = jax.lax.broadcasted_iota(jnp.int32, (_H, _M, _N), 1)
    consts = (Amat, BD, bmask, miota)
    for i in range(blk):
        _one_item(x_ref[i], g1_ref, be1_ref, g2_ref, be2_ref, Wf_ref,
                  bf_ref, Wv_ref, bv_ref, sa_ref, sb_ref, Wp_ref, bp_ref,
                  W1_ref, b1_ref, W2_ref, b2_ref, o_ref, i, consts)


def _one_item(xb, g1_ref, be1_ref, g2_ref, be2_ref, Wf_ref, bf_ref,
              Wv_ref, bv_ref, sa_ref, sb_ref, Wp_ref, bp_ref,
              W1_ref, b1_ref, W2_ref, b2_ref, o_ref, i, consts):
    f32 = jnp.float32
    Amat, BD, bmask, miota = consts

    xn = _ln(xb, g1_ref[...], be1_ref[...])

    xf = _dot_t(xn, Wf_ref[...]) + bf_ref[...]
    vv = _dot_t(xn, Wv_ref[...]) + bv_ref[...]

    cf = _dot(Amat, xf)
    vc = _dot(Amat, vv)

    xnrm = xf * jax.lax.rsqrt(jnp.maximum(_dot(xf * xf, BD), 1e-24))
    cnrm = cf * jax.lax.rsqrt(jnp.maximum(_dot(cf * cf, BD), 1e-24))

    cfs = jnp.concatenate([cnrm] * _H, axis=0) * bmask

    dotp = _dot_t(cfs, xnrm)
    dist = jnp.sqrt(jnp.maximum(2.0 - 2.0 * dotp, 0.0))
    sim = jnp.exp(-dist)
    alpha = sa_ref[0, 0]
    beta = sb_ref[0, 0]
    sraw = beta + alpha * sim
    s = jnp.where(sraw >= 0, sraw, 0.2 * sraw)

    s3 = s.reshape(_H, _M, _N)
    smax = jnp.max(s3, axis=1, keepdims=True)
    cand = jnp.where(s3 >= smax, miota, _M)
    win = jnp.min(cand, axis=1, keepdims=True)
    maskf = jnp.where(miota == win, f32(1.0), f32(0.0))
    masked = (s3 * maskf).reshape(_HM, _N)
    cnt = jnp.sum(maskf, axis=2).reshape(_HM, 1)

    seg = _dot(masked, vv)
    vcs = jnp.concatenate([vc] * _H, axis=0)
    outs = ((seg + vcs) / (cnt + 1.0)) * bmask

    y = jax.lax.dot_general(masked, outs, (((0,), (0,)), ((), ())),
                            preferred_element_type=f32)

    x1 = xb + _dot_t(y, Wp_ref[...]) + bp_ref[...]

    h = _ln(x1, g2_ref[...], be2_ref[...])
    h = _dot_t(h, W1_ref[...]) + b1_ref[...]
    h = h * 0.5 * (1.0 + jax.lax.erf(h * (1.0 / math.sqrt(2.0))))
    h = _dot_t(h, W2_ref[...]) + b2_ref[...]
    o_ref[i] = x1 + h


@jax.jit
def kernel(x, gamma1, beta1, gamma2, beta2, Wf, bf, Wv, bv, sim_alpha,
           sim_beta, Wproj, bproj, W1, b1, W2, b2):
    B = x.shape[0]
    blk = 4
    row = lambda a: a.reshape(1, -1)
    full = lambda shape: pl.BlockSpec(shape, lambda b: (0,) * len(shape))
    args = (
        x,
        row(gamma1), row(beta1), row(gamma2), row(beta2),
        Wf, row(bf), Wv, row(bv),
        sim_alpha.reshape(1, 1), sim_beta.reshape(1, 1),
        Wproj, row(bproj), W1, row(b1), W2, row(b2),
    )
    in_specs = [pl.BlockSpec((blk, _N, _D), lambda b: (b, 0, 0))]
    in_specs += [full(a.shape) for a in args[1:]]
    return pl.pallas_call(
        functools.partial(_fused, blk=blk),
        grid=(B // blk,),
        in_specs=in_specs,
        out_specs=pl.BlockSpec((blk, _N, _D), lambda b: (b, 0, 0)),
        out_shape=jax.ShapeDtypeStruct((B, _N, _D), jnp.float32),
        compiler_params=pltpu.CompilerParams(
            dimension_semantics=("parallel",)),
    )(*args)


# speedup vs baseline: 1.1144x; 1.1144x over previous
import functools
import math

import jax
import jax.numpy as jnp
from jax.experimental import pallas as pl
from jax.experimental.pallas import tpu as pltpu

_P = 24
_D = 384
_H = 8
_HD = 24
_HC = _H * _HD
_M = 16
_N = _P * _P
_HID = _D * 4
_HM = _H * _M


def _ln(x, g, b):
    mu = jnp.mean(x, axis=1, keepdims=True)
    xc = x - mu
    var = jnp.mean(xc * xc, axis=1, keepdims=True)
    return xc * jax.lax.rsqrt(var + 1e-5) * g + b


def _dot_t(a, b):
    return jax.lax.dot_general(a, b, (((1,), (1,)), ((), ())),
                               preferred_element_type=jnp.float32)


def _dot(a, b):
    return jax.lax.dot_general(a, b, (((1,), (0,)), ((), ())),
                               preferred_element_type=jnp.float32)


def _fused(x_ref, g1_ref, be1_ref, g2_ref, be2_ref, Wf_ref, bf_ref,
           Wv_ref, bv_ref, sa_ref, sb_ref, Wp_ref, bp_ref,
           W1_ref, b1_ref, W2_ref, b2_ref, o_ref, *, blk):
    f32 = jnp.float32
    ncol = jax.lax.broadcasted_iota(jnp.int32, (_M, _N), 1)
    mrow = jax.lax.broadcasted_iota(jnp.int32, (_M, _N), 0)
    p1 = ncol // _P
    p2 = ncol - p1 * _P
    mid = (p1 // 6) * 4 + (p2 // 6)
    Amat = jnp.where(mid == mrow, f32(1.0 / 36.0), f32(0.0))
    r0 = jax.lax.broadcasted_iota(jnp.int32, (_HC, _HC), 0)
    c0 = jax.lax.broadcasted_iota(jnp.int32, (_HC, _HC), 1)
    BD = jnp.where(r0 // _HD == c0 // _HD, f32(1.0), f32(0.0))
    rowh = jax.lax.broadcasted_iota(jnp.int32, (_HM, _HC), 0) // _M
    colh = jax.lax.broadcasted_iota(jnp.int32, (_HM, _HC), 1) // _HD
    bmask = jnp.where(rowh == colh, f32(1.0), f32(0.0))
    miota = jax.lax.broadcasted_iota(jnp.int32, (_H, _M, _N), 1)
    consts = (Amat, BD, bmask, miota)
    for i in range(blk):
        _one_item(x_ref[i], g1_ref, be1_ref, g2_ref, be2_ref, Wf_ref,
                  bf_ref, Wv_ref, bv_ref, sa_ref, sb_ref, Wp_ref, bp_ref,
                  W1_ref, b1_ref, W2_ref, b2_ref, o_ref, i, consts)


def _one_item(xb, g1_ref, be1_ref, g2_ref, be2_ref, Wf_ref, bf_ref,
              Wv_ref, bv_ref, sa_ref, sb_ref, Wp_ref, bp_ref,
              W1_ref, b1_ref, W2_ref, b2_ref, o_ref, i, consts):
    f32 = jnp.float32
    Amat, BD, bmask, miota = consts

    xn = _ln(xb, g1_ref[...], be1_ref[...])

    xf = _dot_t(xn, Wf_ref[...]) + bf_ref[...]
    vv = _dot_t(xn, Wv_ref[...]) + bv_ref[...]

    cf = _dot(Amat, xf)
    vc = _dot(Amat, vv)

    xnrm = xf * jax.lax.rsqrt(jnp.maximum(_dot(xf * xf, BD), 1e-24))
    cnrm = cf * jax.lax.rsqrt(jnp.maximum(_dot(cf * cf, BD), 1e-24))

    cfs = jnp.concatenate([cnrm] * _H, axis=0) * bmask

    dotp = _dot_t(cfs, xnrm)
    dist = jnp.sqrt(jnp.maximum(2.0 - 2.0 * dotp, 0.0))
    sim = jnp.exp(-dist)
    alpha = sa_ref[0, 0]
    beta = sb_ref[0, 0]
    sraw = beta + alpha * sim
    s = jnp.where(sraw >= 0, sraw, 0.2 * sraw)

    s3 = s.reshape(_H, _M, _N)
    smax = jnp.max(s3, axis=1, keepdims=True)
    cand = jnp.where(s3 >= smax, miota, _M)
    win = jnp.min(cand, axis=1, keepdims=True)
    maskf = jnp.where(miota == win, f32(1.0), f32(0.0))
    masked = (s3 * maskf).reshape(_HM, _N)
    cnt = jnp.sum(maskf, axis=2).reshape(_HM, 1)

    seg = _dot(masked, vv)
    vcs = jnp.concatenate([vc] * _H, axis=0)
    outs = ((seg + vcs) / (cnt + 1.0)) * bmask

    y = jax.lax.dot_general(masked, outs, (((0,), (0,)), ((), ())),
                            preferred_element_type=f32)

    x1 = xb + _dot_t(y, Wp_ref[...]) + bp_ref[...]

    h = _ln(x1, g2_ref[...], be2_ref[...])
    h = _dot_t(h, W1_ref[...]) + b1_ref[...]
    h = h * 0.5 * (1.0 + jax.lax.erf(h * (1.0 / math.sqrt(2.0))))
    h = _dot_t(h, W2_ref[...]) + b2_ref[...]
    o_ref[i] = x1 + h


@jax.jit
def kernel(x, gamma1, beta1, gamma2, beta2, Wf, bf, Wv, bv, sim_alpha,
           sim_beta, Wproj, bproj, W1, b1, W2, b2):
    B = x.shape[0]
    blk = 4
    row = lambda a: a.reshape(1, -1)
    full = lambda shape: pl.BlockSpec(shape, lambda b: (0,) * len(shape))
    args = (
        x,
        row(gamma1), row(beta1), row(gamma2), row(beta2),
        Wf, row(bf), Wv, row(bv),
        sim_alpha.reshape(1, 1), sim_beta.reshape(1, 1),
        Wproj, row(bproj), W1, row(b1), W2, row(b2),
    )
    in_specs = [pl.BlockSpec((blk, _N, _D), lambda b: (b, 0, 0))]
    in_specs += [full(a.shape) for a in args[1:]]
    return pl.pallas_call(
        functools.partial(_fused, blk=blk),
        grid=(B // blk,),
        in_specs=in_specs,
        out_specs=pl.BlockSpec((blk, _N, _D), lambda b: (b, 0, 0)),
        out_shape=jax.ShapeDtypeStruct((B, _N, _D), jnp.float32),
        compiler_params=pltpu.CompilerParams(
            dimension_semantics=("parallel",)),
    )(*args)


# speedup vs baseline: 1.1629x; 1.0435x over previous
import functools
import math

import jax
import jax.numpy as jnp
from jax.experimental import pallas as pl
from jax.experimental.pallas import tpu as pltpu

_P = 24
_D = 384
_H = 8
_HD = 24
_HC = _H * _HD
_M = 16
_N = _P * _P
_HID = _D * 4
_HM = _H * _M


def _ln(x, g, b):
    mu = jnp.mean(x, axis=1, keepdims=True)
    xc = x - mu
    var = jnp.mean(xc * xc, axis=1, keepdims=True)
    return xc * jax.lax.rsqrt(var + 1e-5) * g + b


def _dot_t(a, b):
    return jax.lax.dot_general(a, b, (((1,), (1,)), ((), ())),
                               preferred_element_type=jnp.float32)


def _dot(a, b):
    return jax.lax.dot_general(a, b, (((1,), (0,)), ((), ())),
                               preferred_element_type=jnp.float32)


def _fused(x_ref, g1_ref, be1_ref, g2_ref, be2_ref, Wf_ref, bf_ref,
           Wv_ref, bv_ref, sa_ref, sb_ref, Wp_ref, bp_ref,
           W1_ref, b1_ref, W2_ref, b2_ref, o_ref, *, blk):
    f32 = jnp.float32
    ncol = jax.lax.broadcasted_iota(jnp.int32, (_M, _N), 1)
    mrow = jax.lax.broadcasted_iota(jnp.int32, (_M, _N), 0)
    p1 = ncol // _P
    p2 = ncol - p1 * _P
    mid = (p1 // 6) * 4 + (p2 // 6)
    Amat = jnp.where(mid == mrow, f32(1.0 / 36.0), f32(0.0))
    r0 = jax.lax.broadcasted_iota(jnp.int32, (_HC, _HC), 0)
    c0 = jax.lax.broadcasted_iota(jnp.int32, (_HC, _HC), 1)
    BD = jnp.where(r0 // _HD == c0 // _HD, f32(1.0), f32(0.0))
    rowh = jax.lax.broadcasted_iota(jnp.int32, (_HM, _HC), 0) // _M
    colh = jax.lax.broadcasted_iota(jnp.int32, (_HM, _HC), 1) // _HD
    bmask = jnp.where(rowh == colh, f32(1.0), f32(0.0))
    miota = jax.lax.broadcasted_iota(jnp.int32, (_H, _M, _N), 1)

    xall = x_ref[...].reshape(blk * _N, _D)
    xn = _ln(xall, g1_ref[...], be1_ref[...])
    XF = _dot_t(xn, Wf_ref[...]) + bf_ref[...]
    VV = _dot_t(xn, Wv_ref[...]) + bv_ref[...]
    XN2 = XF * jax.lax.rsqrt(jnp.maximum(_dot(XF * XF, BD), 1e-24))

    ys = []
    for i in range(blk):
        xf = XF[i * _N:(i + 1) * _N]
        vv = VV[i * _N:(i + 1) * _N]
        xnrm = XN2[i * _N:(i + 1) * _N]

        cf = _dot(Amat, xf)
        vc = _dot(Amat, vv)
        cnrm = cf * jax.lax.rsqrt(jnp.maximum(_dot(cf * cf, BD), 1e-24))

        cfs = jnp.concatenate([cnrm] * _H, axis=0) * bmask

        dotp = _dot_t(cfs, xnrm)
        dist = jnp.sqrt(jnp.maximum(2.0 - 2.0 * dotp, 0.0))
        sim = jnp.exp(-dist)
        sraw = sb_ref[0, 0] + sa_ref[0, 0] * sim
        s = jnp.where(sraw >= 0, sraw, 0.2 * sraw)

        s3 = s.reshape(_H, _M, _N)
        smax = jnp.max(s3, axis=1, keepdims=True)
        cand = jnp.where(s3 >= smax, miota, _M)
        win = jnp.min(cand, axis=1, keepdims=True)
        maskf = jnp.where(miota == win, f32(1.0), f32(0.0))
        masked = (s3 * maskf).reshape(_HM, _N)
        cnt = jnp.sum(maskf, axis=2).reshape(_HM, 1)

        seg = _dot(masked, vv)
        vcs = jnp.concatenate([vc] * _H, axis=0)
        outs = ((seg + vcs) / (cnt + 1.0)) * bmask

        ys.append(jax.lax.dot_general(
            masked, outs, (((0,), (0,)), ((), ())),
            preferred_element_type=f32))

    Y = jnp.concatenate(ys, axis=0)
    x1 = xall + _dot_t(Y, Wp_ref[...]) + bp_ref[...]
    h = _ln(x1, g2_ref[...], be2_ref[...])
    h = _dot_t(h, W1_ref[...]) + b1_ref[...]
    h = h * 0.5 * (1.0 + jax.lax.erf(h * (1.0 / math.sqrt(2.0))))
    h = _dot_t(h, W2_ref[...]) + b2_ref[...]
    o_ref[...] = (x1 + h).reshape(blk, _N, _D)


@jax.jit
def kernel(x, gamma1, beta1, gamma2, beta2, Wf, bf, Wv, bv, sim_alpha,
           sim_beta, Wproj, bproj, W1, b1, W2, b2):
    B = x.shape[0]
    blk = 4
    row = lambda a: a.reshape(1, -1)
    full = lambda shape: pl.BlockSpec(shape, lambda b: (0,) * len(shape))
    args = (
        x,
        row(gamma1), row(beta1), row(gamma2), row(beta2),
        Wf, row(bf), Wv, row(bv),
        sim_alpha.reshape(1, 1), sim_beta.reshape(1, 1),
        Wproj, row(bproj), W1, row(b1), W2, row(b2),
    )
    in_specs = [pl.BlockSpec((blk, _N, _D), lambda b: (b, 0, 0))]
    in_specs += [full(a.shape) for a in args[1:]]
    return pl.pallas_call(
        functools.partial(_fused, blk=blk),
        grid=(B // blk,),
        in_specs=in_specs,
        out_specs=pl.BlockSpec((blk, _N, _D), lambda b: (b, 0, 0)),
        out_shape=jax.ShapeDtypeStruct((B, _N, _D), jnp.float32),
        compiler_params=pltpu.CompilerParams(
            dimension_semantics=("parallel",)),
    )(*args)
